# baseline (speedup 1.0000x reference)
import dataclasses
import functools

import jax
import jax.numpy as jnp
from jax import lax
from jax.experimental import pallas as pl
from jax.experimental.pallas import tpu as pltpu
from jax.experimental.pallas import tpu_sc as plsc

N = 10000
E = 320000
F = 128
NCLASS = 112
NC, NS = 2, 16
NW = NC * NS
EPT = E // NW
CH = 80
NCH = EPT // CH
IBLK = 5
NBAT = NCH // IBLK
NBUF = 4
SB = 2000
NSB = EPT // SB
ZROWS = 40
NZBLK = N // ZROWS
ZBATCH = (NZBLK + NS - 1) // NS
NB = 1000
GRID = N // NB

_SC_MESH = dict(core_axis_name="c", subcore_axis_name="s",
                num_cores=NC, num_subcores=NS)

_SC_PARAMS = pltpu.CompilerParams()
for _f, _v in (("needs_layout_passes", False), ("use_tc_tiling_on_sc", False)):
    if _f in pltpu.CompilerParams.__dataclass_fields__:
        _SC_PARAMS = dataclasses.replace(_SC_PARAMS, **{_f: _v})


def _zero_acc(sid, rowbuf, acc_h):
    zv = jnp.zeros((16,), jnp.float32)

    @pl.loop(0, ZROWS)
    def _(r):
        for k in range(F // 16):
            rowbuf[r, pl.ds(k * 16, 16)] = zv

    @pl.loop(0, ZBATCH)
    def _(t):
        blk = sid + t * NS

        @pl.when(blk < NZBLK)
        def _():
            pltpu.sync_copy(rowbuf.at[pl.ds(0, ZROWS), :],
                            acc_h.at[pl.ds(blk * ZROWS, ZROWS), :])


def _acc_out(cid, sid, acc_h, out_h):
    @pl.loop(0, ZBATCH)
    def _(t):
        blk = sid + t * NS

        @pl.when(blk < NZBLK)
        def _():
            sl = pl.ds(blk * ZROWS, ZROWS)
            pltpu.sync_copy(acc_h.at[sl, :], out_h.at[cid].at[sl, :])


def _scalar_sc_body(as_hbm, ad_hbm, src_hbm, dst_hbm, ex_out, den_out, deg_out,
                    as_tab, ad_tab, den_tab, deg_tab, srcv, dstv, exb):
    cid = lax.axis_index("c")
    sid = lax.axis_index("s")
    wid = cid * NS + sid
    ebase = wid * EPT
    zv = jnp.zeros((16,), jnp.float32)
    one16 = jnp.ones((16,), jnp.float32)

    @pl.loop(0, N // 16)
    def _(r):
        den_tab[pl.ds(r * 16, 16)] = zv
        deg_tab[pl.ds(r * 16, 16)] = zv

    pltpu.sync_copy(as_hbm, as_tab)
    pltpu.sync_copy(ad_hbm, ad_tab)

    @pl.loop(0, NSB)
    def _(k):
        base = ebase + k * SB
        pltpu.sync_copy(src_hbm.at[pl.ds(base, SB)], srcv)
        pltpu.sync_copy(dst_hbm.at[pl.ds(base, SB)], dstv)

        @pl.loop(0, SB // 16)
        def _(g):
            s16 = srcv[pl.ds(g * 16, 16)]
            d16 = dstv[pl.ds(g * 16, 16)]
            s = plsc.load_gather(as_tab, [s16]) + plsc.load_gather(ad_tab, [d16])
            e = jnp.maximum(s, 0.2 * s)
            ex = jnp.exp(e)
            exb[pl.ds(g * 16, 16)] = ex
            plsc.addupdate_scatter(den_tab, [d16], ex)
            plsc.addupdate_scatter(deg_tab, [d16], one16)

        pltpu.sync_copy(exb, ex_out.at[pl.ds(base, SB)])

    pltpu.sync_copy(den_tab, den_out.at[wid])
    pltpu.sync_copy(deg_tab, deg_out.at[wid])


def _scalar2_sc_body(as_hbm, ad_hbm, dsi_hbm, src_hbm, dst_hbm,
                     ex_out, den_out, w_out,
                     as_tab, ad_tab, dsi_tab, den_tab, srcv, dstv, exb, wb):
    cid = lax.axis_index("c")
    sid = lax.axis_index("s")
    wid = cid * NS + sid
    ebase = wid * EPT
    zv = jnp.zeros((16,), jnp.float32)

    @pl.loop(0, N // 16)
    def _(r):
        den_tab[pl.ds(r * 16, 16)] = zv

    pltpu.sync_copy(as_hbm, as_tab)
    pltpu.sync_copy(ad_hbm, ad_tab)
    pltpu.sync_copy(dsi_hbm, dsi_tab)

    @pl.loop(0, NSB)
    def _(k):
        base = ebase + k * SB
        pltpu.sync_copy(src_hbm.at[pl.ds(base, SB)], srcv)
        pltpu.sync_copy(dst_hbm.at[pl.ds(base, SB)], dstv)

        @pl.loop(0, SB // 16)
        def _(g):
            s16 = srcv[pl.ds(g * 16, 16)]
            d16 = dstv[pl.ds(g * 16, 16)]
            sv = plsc.load_gather(as_tab, [s16]) + plsc.load_gather(ad_tab, [d16])
            e = jnp.maximum(sv, 0.2 * sv)
            ex = jnp.exp(e)
            exb[pl.ds(g * 16, 16)] = ex
            plsc.addupdate_scatter(den_tab, [d16], ex)
            w = plsc.load_gather(dsi_tab, [s16]) * plsc.load_gather(dsi_tab, [d16])
            wb[pl.ds(g * 16, 16)] = w

        pltpu.sync_copy(exb, ex_out.at[pl.ds(base, SB)])
        pltpu.sync_copy(wb, w_out.at[pl.ds(base, SB)])

    pltpu.sync_copy(den_tab, den_out.at[wid])


def _scalar2_sc(as_, ad_, dsi, src, dst):
    mesh = plsc.VectorSubcoreMesh(**_SC_MESH)
    return pl.kernel(
        _scalar2_sc_body,
        out_type=[jax.ShapeDtypeStruct((E,), jnp.float32),
                  jax.ShapeDtypeStruct((NW, N), jnp.float32),
                  jax.ShapeDtypeStruct((E,), jnp.float32)],
        mesh=mesh,
        compiler_params=_SC_PARAMS,
        scratch_types=[
            pltpu.VMEM((N,), jnp.float32),
            pltpu.VMEM((N,), jnp.float32),
            pltpu.VMEM((N,), jnp.float32),
            pltpu.VMEM((N,), jnp.float32),
            pltpu.VMEM((SB,), jnp.int32),
            pltpu.VMEM((SB,), jnp.int32),
            pltpu.VMEM((SB,), jnp.float32),
            pltpu.VMEM((SB,), jnp.float32),
        ],
    )(as_, ad_, dsi, src, dst)


def _scalar_sc(as_, ad_, src, dst):
    mesh = plsc.VectorSubcoreMesh(**_SC_MESH)
    return pl.kernel(
        _scalar_sc_body,
        out_type=[jax.ShapeDtypeStruct((E,), jnp.float32),
                  jax.ShapeDtypeStruct((NW, N), jnp.float32),
                  jax.ShapeDtypeStruct((NW, N), jnp.float32)],
        mesh=mesh,
        compiler_params=_SC_PARAMS,
        scratch_types=[
            pltpu.VMEM((N,), jnp.float32),
            pltpu.VMEM((N,), jnp.float32),
            pltpu.VMEM((N,), jnp.float32),
            pltpu.VMEM((N,), jnp.float32),
            pltpu.VMEM((SB,), jnp.int32),
            pltpu.VMEM((SB,), jnp.int32),
            pltpu.VMEM((SB,), jnp.float32),
        ],
    )(as_, ad_, src, dst)


def _attn_row_body(htab, ex2_hbm, src2_hbm, dst2_hbm, out_h,
                   sidx, didx, exv, rb0, rb1, rb2, rb3,
                   gs0, gs1, gs2, gs3, ss0, ss1, ss2, ss3, isem, acc_h):
    cid = lax.axis_index("c")
    sid = lax.axis_index("s")
    cbase = (cid * NS + sid) * NCH
    rowbufs = (rb0, rb1, rb2, rb3)
    gsems = (gs0, gs1, gs2, gs3)
    ssems = (ss0, ss1, ss2, ss3)

    def fr(c):
        return lax.rem(c // IBLK, 2) * IBLK + lax.rem(c, IBLK)

    def row_ref(ref, c):
        return ref.at[pl.ds(fr(c), 1)].at[0]

    def batch_refs(k):
        crow = cbase + k * IBLK
        slot = lax.rem(k, 2)
        sl = pl.ds(slot * IBLK, IBLK)
        return ((src2_hbm.at[pl.ds(crow, IBLK), :], sidx.at[sl, :]),
                (dst2_hbm.at[pl.ds(crow, IBLK), :], didx.at[sl, :]),
                (ex2_hbm.at[pl.ds(crow, IBLK), :], exv.at[sl, :]))

    def load_batch_sync(k):
        for src_r, dst_r in batch_refs(k):
            pltpu.sync_copy(src_r, dst_r)

    def load_batch_start(k):
        for src_r, dst_r in batch_refs(k):
            pltpu.async_copy(src_r, dst_r, isem)

    def load_batch_wait(k):
        for src_r, dst_r in batch_refs(k):
            pltpu.make_async_copy(src_r, dst_r, isem).wait()

    def g_start(c, b):
        pltpu.async_copy(htab.at[row_ref(sidx, c)], rowbufs[b], gsems[b])

    def g_wait(b):
        pltpu.make_async_copy(htab.at[sidx.at[pl.ds(0, 1)].at[0]], rowbufs[b],
                              gsems[b]).wait()

    def s_start(c, b):
        pltpu.async_copy(rowbufs[b], acc_h.at[row_ref(didx, c)],
                         ssems[b], add=True)

    def s_wait(b):
        pltpu.make_async_copy(rowbufs[b], acc_h.at[didx.at[pl.ds(0, 1)].at[0]],
                              ssems[b]).wait()

    load_batch_sync(0)
    load_batch_start(1)
    g_start(0, 0)
    g_start(1, 1)
    _zero_acc(sid, rb2, acc_h)
    plsc.subcore_barrier()

    @pl.loop(0, NCH)
    def _(c):
        @pl.when((lax.rem(c + 2, IBLK) == 0) & (c + 2 < NCH))
        def _():
            load_batch_wait((c + 2) // IBLK)

        frc = fr(c)

        for bb in range(NBUF):
            @pl.when(lax.rem(c, NBUF) == bb)
            def _():
                g_wait(bb)

        @pl.when(c >= 2)
        def _():
            for b2 in range(NBUF):
                @pl.when(lax.rem(c - 2, NBUF) == b2)
                def _():
                    s_wait(b2)

        @pl.when(c + 2 < NCH)
        def _():
            for b2 in range(NBUF):
                @pl.when(lax.rem(c + 2, NBUF) == b2)
                def _():
                    g_start(c + 2, b2)

        @pl.when((lax.rem(c, IBLK) == 2) & (c >= IBLK) & (c // IBLK + 1 < NBAT))
        def _():
            load_batch_start(c // IBLK + 1)

        frc16 = jnp.full((16,), frc, jnp.int32)

        for bb in range(NBUF):
            @pl.when(lax.rem(c, NBUF) == bb)
            def _():
                rb = rowbufs[bb]

                @plsc.parallel_loop(0, CH, unroll=8)
                def _(r):
                    ws = plsc.load_gather(exv,
                                          [frc16, jnp.full((16,), r, jnp.int32)])
                    for q in range(F // 16):
                        rb[r, pl.ds(q * 16, 16)] = rb[r, pl.ds(q * 16, 16)] * ws

                s_start(c, bb)

    for b2 in range(NBUF):
        @pl.when(lax.rem(NCH - 2, NBUF) == b2)
        def _():
            s_wait(b2)

        @pl.when(lax.rem(NCH - 1, NBUF) == b2)
        def _():
            s_wait(b2)

    plsc.subcore_barrier()
    _acc_out(cid, sid, acc_h, out_h)


def _attn_row_sc(htab, ex2, src2, dst2):
    mesh = plsc.VectorSubcoreMesh(**_SC_MESH)
    return pl.kernel(
        _attn_row_body,
        out_type=jax.ShapeDtypeStruct((NC, N, F), jnp.float32),
        mesh=mesh,
        compiler_params=_SC_PARAMS,
        scratch_types=[
            pltpu.VMEM((2 * IBLK, CH), jnp.int32),
            pltpu.VMEM((2 * IBLK, CH), jnp.int32),
            pltpu.VMEM((2 * IBLK, CH), jnp.float32),
            pltpu.VMEM((CH, F), jnp.float32),
            pltpu.VMEM((CH, F), jnp.float32),
            pltpu.VMEM((CH, F), jnp.float32),
            pltpu.VMEM((CH, F), jnp.float32),
            pltpu.SemaphoreType.DMA,
            pltpu.SemaphoreType.DMA,
            pltpu.SemaphoreType.DMA,
            pltpu.SemaphoreType.DMA,
            pltpu.SemaphoreType.DMA,
            pltpu.SemaphoreType.DMA,
            pltpu.SemaphoreType.DMA,
            pltpu.SemaphoreType.DMA,
            pltpu.SemaphoreType.DMA,
            pltpu.VMEM_SHARED((N, F), jnp.float32),
        ],
    )(htab, ex2, src2, dst2)


_dot = functools.partial(jnp.dot, preferred_element_type=jnp.float32,
                         precision=jax.lax.Precision.HIGHEST)


def _mm_body(x_ref, w_ref, ht_ref, as_ref, ad_ref):
    xb = x_ref[...]
    ht_ref[...] = _dot(xb, w_ref[:, :F])
    aa = _dot(xb, w_ref[:, F:])
    as_ref[...] = aa[:, 0:1]
    ad_ref[...] = aa[:, 1:2]


def _mm(x, waug):
    ht, asv, adv = pl.pallas_call(
        _mm_body,
        grid=(GRID,),
        in_specs=[pl.BlockSpec((NB, F), lambda i: (i, 0)),
                  pl.BlockSpec((F, F + 2), lambda i: (0, 0))],
        out_specs=[pl.BlockSpec((NB, F), lambda i: (i, 0)),
                   pl.BlockSpec((NB, 1), lambda i: (i, 0)),
                   pl.BlockSpec((NB, 1), lambda i: (i, 0))],
        out_shape=[jax.ShapeDtypeStruct((N, F), jnp.float32),
                   jax.ShapeDtypeStruct((N, 1), jnp.float32),
                   jax.ShapeDtypeStruct((N, 1), jnp.float32)],
    )(x, waug)
    return ht, asv.reshape(N), adv.reshape(N)


def _merge_body(acch_ref, den_ref, deg_ref, agg_ref, dsi_ref, stats_ref):
    i = pl.program_id(0)
    a = acch_ref[0] + acch_ref[1]
    agg = a / (den_ref[...] + 1e-16)
    agg_ref[...] = agg
    dsi_ref[...] = lax.rsqrt(jnp.maximum(deg_ref[...], 1.0))

    @pl.when(i == 0)
    def _():
        stats_ref[...] = jnp.zeros_like(stats_ref)

    stats_ref[0:1, :] += jnp.sum(agg, axis=0, keepdims=True)
    stats_ref[1:2, :] += jnp.sum(agg * agg, axis=0, keepdims=True)


def _merge(acc_h, den, deg):
    agg, dsi, stats = pl.pallas_call(
        _merge_body,
        grid=(GRID,),
        in_specs=[pl.BlockSpec((NC, NB, F), lambda i: (0, i, 0)),
                  pl.BlockSpec((NB, 1), lambda i: (i, 0)),
                  pl.BlockSpec((NB, 1), lambda i: (i, 0))],
        out_specs=[pl.BlockSpec((NB, F), lambda i: (i, 0)),
                   pl.BlockSpec((NB, 1), lambda i: (i, 0)),
                   pl.BlockSpec((8, F), lambda i: (0, 0))],
        out_shape=[jax.ShapeDtypeStruct((N, F), jnp.float32),
                   jax.ShapeDtypeStruct((N, 1), jnp.float32),
                   jax.ShapeDtypeStruct((8, F), jnp.float32)],
    )(acc_h, den, deg)
    return agg, dsi.reshape(N), stats


def _bn_mm_body(agg_ref, stats_ref, g_ref, b_ref, w_ref, ht_ref, as_ref, ad_ref):
    st = stats_ref[...]
    m = st[0:1, :] / N
    v = st[1:2, :] / N - m * m
    inv = lax.rsqrt(v + 1e-5)
    xn = g_ref[...] * (agg_ref[...] - m) * inv + b_ref[...]
    h = jnp.where(xn >= 0, xn, 0.01 * xn)
    ht_ref[...] = _dot(h, w_ref[:, :F])
    aa = _dot(h, w_ref[:, F:])
    as_ref[...] = aa[:, 0:1]
    ad_ref[...] = aa[:, 1:2]


def _bn_mm(agg, stats, gamma, beta, waug):
    ht, asv, adv = pl.pallas_call(
        _bn_mm_body,
        grid=(GRID,),
        in_specs=[pl.BlockSpec((NB, F), lambda i: (i, 0)),
                  pl.BlockSpec((8, F), lambda i: (0, 0)),
                  pl.BlockSpec((1, F), lambda i: (0, 0)),
                  pl.BlockSpec((1, F), lambda i: (0, 0)),
                  pl.BlockSpec((F, F + 2), lambda i: (0, 0))],
        out_specs=[pl.BlockSpec((NB, F), lambda i: (i, 0)),
                   pl.BlockSpec((NB, 1), lambda i: (i, 0)),
                   pl.BlockSpec((NB, 1), lambda i: (i, 0))],
        out_shape=[jax.ShapeDtypeStruct((N, F), jnp.float32),
                   jax.ShapeDtypeStruct((N, 1), jnp.float32),
                   jax.ShapeDtypeStruct((N, 1), jnp.float32)],
    )(agg, stats, gamma, beta, waug)
    return ht, asv.reshape(N), adv.reshape(N)


def _bn_mm2_body(agg_ref, stats_ref, g_ref, b_ref, w_ref, st_ref):
    st = stats_ref[...]
    m = st[0:1, :] / N
    v = st[1:2, :] / N - m * m
    inv = lax.rsqrt(v + 1e-5)
    xn = g_ref[...] * (agg_ref[...] - m) * inv + b_ref[...]
    h = jnp.where(xn >= 0, xn, 0.01 * xn)
    st_ref[...] = _dot(h, w_ref[...])


def _bn_mm2(agg, stats, gamma, beta, wpad):
    return pl.pallas_call(
        _bn_mm2_body,
        grid=(GRID,),
        in_specs=[pl.BlockSpec((NB, F), lambda i: (i, 0)),
                  pl.BlockSpec((8, F), lambda i: (0, 0)),
                  pl.BlockSpec((1, F), lambda i: (0, 0)),
                  pl.BlockSpec((1, F), lambda i: (0, 0)),
                  pl.BlockSpec((F, F), lambda i: (0, 0))],
        out_specs=pl.BlockSpec((NB, F), lambda i: (i, 0)),
        out_shape=jax.ShapeDtypeStruct((N, F), jnp.float32),
    )(agg, stats, gamma, beta, wpad)


def _final_body(acc_ref, stab_ref, out_ref):
    a = acc_ref[0] + acc_ref[1]
    out_ref[...] = (0.5 * a[:, :NCLASS] + stab_ref[:, :NCLASS]) / 1.5


def _final(acc, stab):
    return pl.pallas_call(
        _final_body,
        grid=(GRID,),
        in_specs=[pl.BlockSpec((NC, NB, F), lambda i: (0, i, 0)),
                  pl.BlockSpec((NB, F), lambda i: (i, 0))],
        out_specs=pl.BlockSpec((NB, NCLASS), lambda i: (i, 0)),
        out_shape=jax.ShapeDtypeStruct((N, NCLASS), jnp.float32),
    )(acc, stab)


def kernel(x, edge_index, W1, a1_src, a1_dst, W2, a2_src, a2_dst, gamma, beta, W_res):
    src = edge_index[0]
    dst = edge_index[1]
    w1aug = jnp.concatenate([W1, (W1 @ a1_src)[:, None], (W1 @ a1_dst)[:, None]], axis=1)
    w2aug = jnp.concatenate([W2, (W2 @ a2_src)[:, None], (W2 @ a2_dst)[:, None]], axis=1)
    wres_pad = jnp.concatenate([W_res, jnp.zeros((F, F - NCLASS), jnp.float32)], axis=1)
    g2 = gamma.reshape(1, F)
    b2 = beta.reshape(1, F)

    src2 = src.reshape(E // CH, CH)
    dst2 = dst.reshape(E // CH, CH)

    htab1, as1, ad1 = _mm(x, w1aug)
    ex1, den1, deg1 = _scalar_sc(as1, ad1, src, dst)
    acc1h = _attn_row_sc(htab1, ex1.reshape(E // CH, CH), src2, dst2)
    agg1, dsi, stats1 = _merge(acc1h, den1.sum(0).reshape(N, 1), deg1.sum(0).reshape(N, 1))
    htab2, as2, ad2 = _bn_mm(agg1, stats1, g2, b2, w2aug)
    exl2, den2, wgc = _scalar2_sc(as2, ad2, dsi, src, dst)
    acc2h = _attn_row_sc(htab2, exl2.reshape(E // CH, CH), src2, dst2)
    agg2, _, stats2 = _merge(acc2h, den2.sum(0).reshape(N, 1), jnp.ones((N, 1), jnp.float32))
    stab = _bn_mm2(agg2, stats2, g2, b2, wres_pad)
    acc3 = _attn_row_sc(stab, wgc.reshape(E // CH, CH), src2, dst2)
    return _final(acc3, stab)


# speedup vs baseline: 1.0077x; 1.0077x over previous
import dataclasses
import functools

import jax
import jax.numpy as jnp
from jax import lax
from jax.experimental import pallas as pl
from jax.experimental.pallas import tpu as pltpu
from jax.experimental.pallas import tpu_sc as plsc

N = 10000
E = 320000
F = 128
NCLASS = 112
NC, NS = 2, 16
NW = NC * NS
EPT = E // NW
CH = 80
NCH = EPT // CH
IBLK = 5
NBAT = NCH // IBLK
NBUF = 4
SB = 2000
NSB = EPT // SB
ZROWS = 40
NZBLK = N // ZROWS
ZBATCH = (NZBLK + NS - 1) // NS
NB = 1000
GRID = N // NB

_SC_MESH = dict(core_axis_name="c", subcore_axis_name="s",
                num_cores=NC, num_subcores=NS)

_SC_PARAMS = pltpu.CompilerParams()
for _f, _v in (("needs_layout_passes", False), ("use_tc_tiling_on_sc", False)):
    if _f in pltpu.CompilerParams.__dataclass_fields__:
        _SC_PARAMS = dataclasses.replace(_SC_PARAMS, **{_f: _v})


def _zero_acc(sid, rowbuf, acc_h):
    zv = jnp.zeros((16,), jnp.float32)

    @pl.loop(0, ZROWS)
    def _(r):
        for k in range(F // 16):
            rowbuf[r, pl.ds(k * 16, 16)] = zv

    @pl.loop(0, ZBATCH)
    def _(t):
        blk = sid + t * NS

        @pl.when(blk < NZBLK)
        def _():
            pltpu.sync_copy(rowbuf.at[pl.ds(0, ZROWS), :],
                            acc_h.at[pl.ds(blk * ZROWS, ZROWS), :])


def _acc_out(cid, sid, acc_h, out_h):
    @pl.loop(0, ZBATCH)
    def _(t):
        blk = sid + t * NS

        @pl.when(blk < NZBLK)
        def _():
            sl = pl.ds(blk * ZROWS, ZROWS)
            pltpu.sync_copy(acc_h.at[sl, :], out_h.at[cid].at[sl, :])


def _scalar_sc_body(as_hbm, ad_hbm, src_hbm, dst_hbm, ex_out, den_out, deg_out,
                    as_tab, ad_tab, den_tab, deg_tab, srcv, dstv, exb):
    cid = lax.axis_index("c")
    sid = lax.axis_index("s")
    wid = cid * NS + sid
    ebase = wid * EPT
    zv = jnp.zeros((16,), jnp.float32)
    one16 = jnp.ones((16,), jnp.float32)

    @pl.loop(0, N // 16)
    def _(r):
        den_tab[pl.ds(r * 16, 16)] = zv
        deg_tab[pl.ds(r * 16, 16)] = zv

    pltpu.sync_copy(as_hbm, as_tab)
    pltpu.sync_copy(ad_hbm, ad_tab)

    @pl.loop(0, NSB)
    def _(k):
        base = ebase + k * SB
        pltpu.sync_copy(src_hbm.at[pl.ds(base, SB)], srcv)
        pltpu.sync_copy(dst_hbm.at[pl.ds(base, SB)], dstv)

        @pl.loop(0, SB // 16)
        def _(g):
            s16 = srcv[pl.ds(g * 16, 16)]
            d16 = dstv[pl.ds(g * 16, 16)]
            s = plsc.load_gather(as_tab, [s16]) + plsc.load_gather(ad_tab, [d16])
            e = jnp.maximum(s, 0.2 * s)
            ex = jnp.exp(e)
            exb[pl.ds(g * 16, 16)] = ex
            plsc.addupdate_scatter(den_tab, [d16], ex)
            plsc.addupdate_scatter(deg_tab, [d16], one16)

        pltpu.sync_copy(exb, ex_out.at[pl.ds(base, SB)])

    pltpu.sync_copy(den_tab, den_out.at[wid])
    pltpu.sync_copy(deg_tab, deg_out.at[wid])


def _scalar2_sc_body(as_hbm, ad_hbm, dsi_hbm, src_hbm, dst_hbm,
                     ex_out, den_out, w_out,
                     as_tab, ad_tab, dsi_tab, den_tab, srcv, dstv, exb, wb):
    cid = lax.axis_index("c")
    sid = lax.axis_index("s")
    wid = cid * NS + sid
    ebase = wid * EPT
    zv = jnp.zeros((16,), jnp.float32)

    @pl.loop(0, N // 16)
    def _(r):
        den_tab[pl.ds(r * 16, 16)] = zv

    pltpu.sync_copy(as_hbm, as_tab)
    pltpu.sync_copy(ad_hbm, ad_tab)
    pltpu.sync_copy(dsi_hbm, dsi_tab)

    @pl.loop(0, NSB)
    def _(k):
        base = ebase + k * SB
        pltpu.sync_copy(src_hbm.at[pl.ds(base, SB)], srcv)
        pltpu.sync_copy(dst_hbm.at[pl.ds(base, SB)], dstv)

        @pl.loop(0, SB // 16)
        def _(g):
            s16 = srcv[pl.ds(g * 16, 16)]
            d16 = dstv[pl.ds(g * 16, 16)]
            sv = plsc.load_gather(as_tab, [s16]) + plsc.load_gather(ad_tab, [d16])
            e = jnp.maximum(sv, 0.2 * sv)
            ex = jnp.exp(e)
            exb[pl.ds(g * 16, 16)] = ex
            plsc.addupdate_scatter(den_tab, [d16], ex)
            w = plsc.load_gather(dsi_tab, [s16]) * plsc.load_gather(dsi_tab, [d16])
            wb[pl.ds(g * 16, 16)] = w

        pltpu.sync_copy(exb, ex_out.at[pl.ds(base, SB)])
        pltpu.sync_copy(wb, w_out.at[pl.ds(base, SB)])

    pltpu.sync_copy(den_tab, den_out.at[wid])


def _scalar2_sc(as_, ad_, dsi, src, dst):
    mesh = plsc.VectorSubcoreMesh(**_SC_MESH)
    return pl.kernel(
        _scalar2_sc_body,
        out_type=[jax.ShapeDtypeStruct((E,), jnp.float32),
                  jax.ShapeDtypeStruct((NW, N), jnp.float32),
                  jax.ShapeDtypeStruct((E,), jnp.float32)],
        mesh=mesh,
        compiler_params=_SC_PARAMS,
        scratch_types=[
            pltpu.VMEM((N,), jnp.float32),
            pltpu.VMEM((N,), jnp.float32),
            pltpu.VMEM((N,), jnp.float32),
            pltpu.VMEM((N,), jnp.float32),
            pltpu.VMEM((SB,), jnp.int32),
            pltpu.VMEM((SB,), jnp.int32),
            pltpu.VMEM((SB,), jnp.float32),
            pltpu.VMEM((SB,), jnp.float32),
        ],
    )(as_, ad_, dsi, src, dst)


def _scalar_sc(as_, ad_, src, dst):
    mesh = plsc.VectorSubcoreMesh(**_SC_MESH)
    return pl.kernel(
        _scalar_sc_body,
        out_type=[jax.ShapeDtypeStruct((E,), jnp.float32),
                  jax.ShapeDtypeStruct((NW, N), jnp.float32),
                  jax.ShapeDtypeStruct((NW, N), jnp.float32)],
        mesh=mesh,
        compiler_params=_SC_PARAMS,
        scratch_types=[
            pltpu.VMEM((N,), jnp.float32),
            pltpu.VMEM((N,), jnp.float32),
            pltpu.VMEM((N,), jnp.float32),
            pltpu.VMEM((N,), jnp.float32),
            pltpu.VMEM((SB,), jnp.int32),
            pltpu.VMEM((SB,), jnp.int32),
            pltpu.VMEM((SB,), jnp.float32),
        ],
    )(as_, ad_, src, dst)


def _attn_row_body(htab, ex2_hbm, src2_hbm, dst2_hbm, out_h,
                   sidx, didx, exv, rb0, rb1, rb2, rb3,
                   gs0, gs1, gs2, gs3, ss0, ss1, ss2, ss3, isem, acc_h):
    cid = lax.axis_index("c")
    sid = lax.axis_index("s")
    cbase = (cid * NS + sid) * NCH
    rowbufs = (rb0, rb1, rb2, rb3)
    gsems = (gs0, gs1, gs2, gs3)
    ssems = (ss0, ss1, ss2, ss3)

    def fr(c):
        return lax.rem(c // IBLK, 2) * IBLK + lax.rem(c, IBLK)

    def row_ref(ref, c):
        return ref.at[pl.ds(fr(c), 1)].at[0]

    def batch_refs(k):
        crow = cbase + k * IBLK
        slot = lax.rem(k, 2)
        sl = pl.ds(slot * IBLK, IBLK)
        return ((src2_hbm.at[pl.ds(crow, IBLK), :], sidx.at[sl, :]),
                (dst2_hbm.at[pl.ds(crow, IBLK), :], didx.at[sl, :]),
                (ex2_hbm.at[pl.ds(crow, IBLK), :], exv.at[sl, :]))

    def load_batch_sync(k):
        for src_r, dst_r in batch_refs(k):
            pltpu.sync_copy(src_r, dst_r)

    def load_batch_start(k):
        for src_r, dst_r in batch_refs(k):
            pltpu.async_copy(src_r, dst_r, isem)

    def load_batch_wait(k):
        for src_r, dst_r in batch_refs(k):
            pltpu.make_async_copy(src_r, dst_r, isem).wait()

    def g_start(c, b):
        pltpu.async_copy(htab.at[row_ref(sidx, c)], rowbufs[b], gsems[b])

    def g_wait(b):
        pltpu.make_async_copy(htab.at[sidx.at[pl.ds(0, 1)].at[0]], rowbufs[b],
                              gsems[b]).wait()

    def s_start(c, b):
        pltpu.async_copy(rowbufs[b], acc_h.at[row_ref(didx, c)],
                         ssems[b], add=True)

    def s_wait(b):
        pltpu.make_async_copy(rowbufs[b], acc_h.at[didx.at[pl.ds(0, 1)].at[0]],
                              ssems[b]).wait()

    load_batch_sync(0)
    load_batch_start(1)
    g_start(0, 0)
    g_start(1, 1)
    _zero_acc(sid, rb2, acc_h)
    plsc.subcore_barrier()

    @pl.loop(0, NCH)
    def _(c):
        @pl.when((lax.rem(c + 2, IBLK) == 0) & (c + 2 < NCH))
        def _():
            load_batch_wait((c + 2) // IBLK)

        frc = fr(c)

        for bb in range(NBUF):
            @pl.when(lax.rem(c, NBUF) == bb)
            def _():
                g_wait(bb)

        @pl.when(c >= 2)
        def _():
            for b2 in range(NBUF):
                @pl.when(lax.rem(c - 2, NBUF) == b2)
                def _():
                    s_wait(b2)

        @pl.when(c + 2 < NCH)
        def _():
            for b2 in range(NBUF):
                @pl.when(lax.rem(c + 2, NBUF) == b2)
                def _():
                    g_start(c + 2, b2)

        @pl.when((lax.rem(c, IBLK) == 2) & (c >= IBLK) & (c // IBLK + 1 < NBAT))
        def _():
            load_batch_start(c // IBLK + 1)

        frc16 = jnp.full((16,), frc, jnp.int32)

        for bb in range(NBUF):
            @pl.when(lax.rem(c, NBUF) == bb)
            def _():
                rb = rowbufs[bb]

                @plsc.parallel_loop(0, CH, unroll=4)
                def _(r):
                    ws = plsc.load_gather(exv,
                                          [frc16, jnp.full((16,), r, jnp.int32)])
                    for q in range(F // 16):
                        rb[r, pl.ds(q * 16, 16)] = rb[r, pl.ds(q * 16, 16)] * ws

                s_start(c, bb)

    for b2 in range(NBUF):
        @pl.when(lax.rem(NCH - 2, NBUF) == b2)
        def _():
            s_wait(b2)

        @pl.when(lax.rem(NCH - 1, NBUF) == b2)
        def _():
            s_wait(b2)

    plsc.subcore_barrier()
    _acc_out(cid, sid, acc_h, out_h)


def _attn_row_sc(htab, ex2, src2, dst2):
    mesh = plsc.VectorSubcoreMesh(**_SC_MESH)
    return pl.kernel(
        _attn_row_body,
        out_type=jax.ShapeDtypeStruct((NC, N, F), jnp.float32),
        mesh=mesh,
        compiler_params=_SC_PARAMS,
        scratch_types=[
            pltpu.VMEM((2 * IBLK, CH), jnp.int32),
            pltpu.VMEM((2 * IBLK, CH), jnp.int32),
            pltpu.VMEM((2 * IBLK, CH), jnp.float32),
            pltpu.VMEM((CH, F), jnp.float32),
            pltpu.VMEM((CH, F), jnp.float32),
            pltpu.VMEM((CH, F), jnp.float32),
            pltpu.VMEM((CH, F), jnp.float32),
            pltpu.SemaphoreType.DMA,
            pltpu.SemaphoreType.DMA,
            pltpu.SemaphoreType.DMA,
            pltpu.SemaphoreType.DMA,
            pltpu.SemaphoreType.DMA,
            pltpu.SemaphoreType.DMA,
            pltpu.SemaphoreType.DMA,
            pltpu.SemaphoreType.DMA,
            pltpu.SemaphoreType.DMA,
            pltpu.VMEM_SHARED((N, F), jnp.float32),
        ],
    )(htab, ex2, src2, dst2)


_dot = functools.partial(jnp.dot, preferred_element_type=jnp.float32,
                         precision=jax.lax.Precision.HIGHEST)


def _mm_body(x_ref, w_ref, ht_ref, as_ref, ad_ref):
    xb = x_ref[...]
    ht_ref[...] = _dot(xb, w_ref[:, :F])
    aa = _dot(xb, w_ref[:, F:])
    as_ref[...] = aa[:, 0:1]
    ad_ref[...] = aa[:, 1:2]


def _mm(x, waug):
    ht, asv, adv = pl.pallas_call(
        _mm_body,
        grid=(GRID,),
        in_specs=[pl.BlockSpec((NB, F), lambda i: (i, 0)),
                  pl.BlockSpec((F, F + 2), lambda i: (0, 0))],
        out_specs=[pl.BlockSpec((NB, F), lambda i: (i, 0)),
                   pl.BlockSpec((NB, 1), lambda i: (i, 0)),
                   pl.BlockSpec((NB, 1), lambda i: (i, 0))],
        out_shape=[jax.ShapeDtypeStruct((N, F), jnp.float32),
                   jax.ShapeDtypeStruct((N, 1), jnp.float32),
                   jax.ShapeDtypeStruct((N, 1), jnp.float32)],
    )(x, waug)
    return ht, asv.reshape(N), adv.reshape(N)


def _merge_body(acch_ref, den_ref, deg_ref, agg_ref, dsi_ref, stats_ref):
    i = pl.program_id(0)
    a = acch_ref[0] + acch_ref[1]
    agg = a / (den_ref[...] + 1e-16)
    agg_ref[...] = agg
    dsi_ref[...] = lax.rsqrt(jnp.maximum(deg_ref[...], 1.0))

    @pl.when(i == 0)
    def _():
        stats_ref[...] = jnp.zeros_like(stats_ref)

    stats_ref[0:1, :] += jnp.sum(agg, axis=0, keepdims=True)
    stats_ref[1:2, :] += jnp.sum(agg * agg, axis=0, keepdims=True)


def _merge(acc_h, den, deg):
    agg, dsi, stats = pl.pallas_call(
        _merge_body,
        grid=(GRID,),
        in_specs=[pl.BlockSpec((NC, NB, F), lambda i: (0, i, 0)),
                  pl.BlockSpec((NB, 1), lambda i: (i, 0)),
                  pl.BlockSpec((NB, 1), lambda i: (i, 0))],
        out_specs=[pl.BlockSpec((NB, F), lambda i: (i, 0)),
                   pl.BlockSpec((NB, 1), lambda i: (i, 0)),
                   pl.BlockSpec((8, F), lambda i: (0, 0))],
        out_shape=[jax.ShapeDtypeStruct((N, F), jnp.float32),
                   jax.ShapeDtypeStruct((N, 1), jnp.float32),
                   jax.ShapeDtypeStruct((8, F), jnp.float32)],
    )(acc_h, den, deg)
    return agg, dsi.reshape(N), stats


def _bn_mm_body(agg_ref, stats_ref, g_ref, b_ref, w_ref, ht_ref, as_ref, ad_ref):
    st = stats_ref[...]
    m = st[0:1, :] / N
    v = st[1:2, :] / N - m * m
    inv = lax.rsqrt(v + 1e-5)
    xn = g_ref[...] * (agg_ref[...] - m) * inv + b_ref[...]
    h = jnp.where(xn >= 0, xn, 0.01 * xn)
    ht_ref[...] = _dot(h, w_ref[:, :F])
    aa = _dot(h, w_ref[:, F:])
    as_ref[...] = aa[:, 0:1]
    ad_ref[...] = aa[:, 1:2]


def _bn_mm(agg, stats, gamma, beta, waug):
    ht, asv, adv = pl.pallas_call(
        _bn_mm_body,
        grid=(GRID,),
        in_specs=[pl.BlockSpec((NB, F), lambda i: (i, 0)),
                  pl.BlockSpec((8, F), lambda i: (0, 0)),
                  pl.BlockSpec((1, F), lambda i: (0, 0)),
                  pl.BlockSpec((1, F), lambda i: (0, 0)),
                  pl.BlockSpec((F, F + 2), lambda i: (0, 0))],
        out_specs=[pl.BlockSpec((NB, F), lambda i: (i, 0)),
                   pl.BlockSpec((NB, 1), lambda i: (i, 0)),
                   pl.BlockSpec((NB, 1), lambda i: (i, 0))],
        out_shape=[jax.ShapeDtypeStruct((N, F), jnp.float32),
                   jax.ShapeDtypeStruct((N, 1), jnp.float32),
                   jax.ShapeDtypeStruct((N, 1), jnp.float32)],
    )(agg, stats, gamma, beta, waug)
    return ht, asv.reshape(N), adv.reshape(N)


def _bn_mm2_body(agg_ref, stats_ref, g_ref, b_ref, w_ref, st_ref):
    st = stats_ref[...]
    m = st[0:1, :] / N
    v = st[1:2, :] / N - m * m
    inv = lax.rsqrt(v + 1e-5)
    xn = g_ref[...] * (agg_ref[...] - m) * inv + b_ref[...]
    h = jnp.where(xn >= 0, xn, 0.01 * xn)
    st_ref[...] = _dot(h, w_ref[...])


def _bn_mm2(agg, stats, gamma, beta, wpad):
    return pl.pallas_call(
        _bn_mm2_body,
        grid=(GRID,),
        in_specs=[pl.BlockSpec((NB, F), lambda i: (i, 0)),
                  pl.BlockSpec((8, F), lambda i: (0, 0)),
                  pl.BlockSpec((1, F), lambda i: (0, 0)),
                  pl.BlockSpec((1, F), lambda i: (0, 0)),
                  pl.BlockSpec((F, F), lambda i: (0, 0))],
        out_specs=pl.BlockSpec((NB, F), lambda i: (i, 0)),
        out_shape=jax.ShapeDtypeStruct((N, F), jnp.float32),
    )(agg, stats, gamma, beta, wpad)


def _final_body(acc_ref, stab_ref, out_ref):
    a = acc_ref[0] + acc_ref[1]
    out_ref[...] = (0.5 * a[:, :NCLASS] + stab_ref[:, :NCLASS]) / 1.5


def _final(acc, stab):
    return pl.pallas_call(
        _final_body,
        grid=(GRID,),
        in_specs=[pl.BlockSpec((NC, NB, F), lambda i: (0, i, 0)),
                  pl.BlockSpec((NB, F), lambda i: (i, 0))],
        out_specs=pl.BlockSpec((NB, NCLASS), lambda i: (i, 0)),
        out_shape=jax.ShapeDtypeStruct((N, NCLASS), jnp.float32),
    )(acc, stab)


def kernel(x, edge_index, W1, a1_src, a1_dst, W2, a2_src, a2_dst, gamma, beta, W_res):
    src = edge_index[0]
    dst = edge_index[1]
    w1aug = jnp.concatenate([W1, (W1 @ a1_src)[:, None], (W1 @ a1_dst)[:, None]], axis=1)
    w2aug = jnp.concatenate([W2, (W2 @ a2_src)[:, None], (W2 @ a2_dst)[:, None]], axis=1)
    wres_pad = jnp.concatenate([W_res, jnp.zeros((F, F - NCLASS), jnp.float32)], axis=1)
    g2 = gamma.reshape(1, F)
    b2 = beta.reshape(1, F)

    src2 = src.reshape(E // CH, CH)
    dst2 = dst.reshape(E // CH, CH)

    htab1, as1, ad1 = _mm(x, w1aug)
    ex1, den1, deg1 = _scalar_sc(as1, ad1, src, dst)
    acc1h = _attn_row_sc(htab1, ex1.reshape(E // CH, CH), src2, dst2)
    agg1, dsi, stats1 = _merge(acc1h, den1.sum(0).reshape(N, 1), deg1.sum(0).reshape(N, 1))
    htab2, as2, ad2 = _bn_mm(agg1, stats1, g2, b2, w2aug)
    exl2, den2, wgc = _scalar2_sc(as2, ad2, dsi, src, dst)
    acc2h = _attn_row_sc(htab2, exl2.reshape(E // CH, CH), src2, dst2)
    agg2, _, stats2 = _merge(acc2h, den2.sum(0).reshape(N, 1), jnp.ones((N, 1), jnp.float32))
    stab = _bn_mm2(agg2, stats2, g2, b2, wres_pad)
    acc3 = _attn_row_sc(stab, wgc.reshape(E // CH, CH), src2, dst2)
    return _final(acc3, stab)
